# baseline (speedup 1.0000x reference)
import functools

import jax
import jax.numpy as jnp
from jax import lax
from jax.experimental import pallas as pl
from jax.experimental.pallas import tpu as pltpu
from jax.experimental.pallas import tpu_sc as plsc

N = 10000
NP = 10240
E = 160000
D = 256
H = 128
NC_OUT = 12
ROWS_PER_TILE = NP // 16
CHUNK = 128
NSTAGE = 4
SCH = 20
EDGES_PER_TILE = NSTAGE * SCH * CHUNK
EP = 16 * EDGES_PER_TILE
BLK = 1024

_SC_PARAMS = pltpu.CompilerParams(needs_layout_passes=False,
                                  internal_scratch_in_bytes=8192)


def _mesh():
    return plsc.VectorSubcoreMesh(core_axis_name="c", subcore_axis_name="s")


def _hist_reduce(s, hist_v, slots_sh, red_v, acc_v):
    pltpu.sync_copy(hist_v, slots_sh.at[s])
    plsc.subcore_barrier()
    pltpu.sync_copy(slots_sh.at[:, pl.ds(s * ROWS_PER_TILE, ROWS_PER_TILE)],
                    red_v)

    @pl.loop(0, ROWS_PER_TILE // 16)
    def _(j):
        v = red_v[0, pl.ds(j * 16, 16)]
        for r in range(1, 16):
            v = v + red_v[r, pl.ds(j * 16, 16)]
        acc_v[pl.ds(j * 16, 16)] = v


def _rsqrt16(x):
    i = plsc.bitcast(x, jnp.int32)
    i = jnp.int32(0x5F3759DF) - lax.shift_right_logical(i, 1)
    y = plsc.bitcast(i, jnp.float32)
    for _ in range(3):
        y = y * (1.5 - 0.5 * x * y * y)
    return y


def _zero_1d(ref, n):
    zeros = jnp.zeros((16,), jnp.float32)

    @pl.loop(0, n // 16)
    def _(i):
        ref[pl.ds(i * 16, 16)] = zeros


def _front_body(idx_hbm, feats_hbm, a12_hbm, nsd_hbm,
                sidx_v, didx_v, hist_v, hist2_v, z_v, red_v, acc_v, acc2_v,
                fe_v, slots_sh, z_sh):
    c = lax.axis_index("c")
    s = lax.axis_index("s")
    stripe = pl.ds(s * ROWS_PER_TILE, ROWS_PER_TILE)
    pltpu.sync_copy(idx_hbm.at[0, s], sidx_v)
    pltpu.sync_copy(idx_hbm.at[1, s], didx_v)
    pltpu.sync_copy(feats_hbm.at[stripe], fe_v)
    _zero_1d(hist_v, NP)
    _zero_1d(hist2_v, NP)
    ones = jnp.ones((16,), jnp.float32)

    @pl.loop(0, EDGES_PER_TILE // 16)
    def _(i):
        plsc.addupdate_scatter(hist_v, [sidx_v[pl.ds(i * 16, 16)]], ones)
        plsc.addupdate_scatter(hist2_v, [didx_v[pl.ds(i * 16, 16)]], ones)

    _hist_reduce(s, hist_v, slots_sh, red_v, acc_v)
    plsc.subcore_barrier()
    _hist_reduce(s, hist2_v, slots_sh, red_v, acc2_v)

    @pl.loop(0, ROWS_PER_TILE // 16)
    def _(j):
        sl = pl.ds(j * 16, 16)
        ns = _rsqrt16(jnp.maximum(acc_v[sl], 1.0))
        acc_v[sl] = ns
        acc2_v[sl] = _rsqrt16(jnp.maximum(acc2_v[sl], 1.0))
        fe_v[sl] = fe_v[sl] * ns

    @pl.when(c == 0)
    def _():
        pltpu.sync_copy(acc_v, nsd_hbm.at[0].at[stripe])

    @pl.when(c == 1)
    def _():
        pltpu.sync_copy(acc2_v, nsd_hbm.at[1].at[stripe])

    pltpu.sync_copy(fe_v, z_sh.at[0].at[stripe])
    pltpu.sync_copy(acc_v, z_sh.at[1].at[stripe])
    plsc.subcore_barrier()
    pltpu.sync_copy(z_sh.at[c], z_v)
    _zero_1d(hist_v, NP)

    @pl.loop(0, EDGES_PER_TILE // 16)
    def _(i):
        sv = sidx_v[pl.ds(i * 16, 16)]
        dv = didx_v[pl.ds(i * 16, 16)]
        val = plsc.load_gather(z_v, [sv])
        plsc.addupdate_scatter(hist_v, [dv], val)

    _hist_reduce(s, hist_v, slots_sh, red_v, acc_v)
    pltpu.sync_copy(acc_v, a12_hbm.at[c].at[stripe])


@jax.jit
def _front_call(idx_a, feats_pad):
    return pl.kernel(
        _front_body,
        out_type=(jax.ShapeDtypeStruct((2, NP), jnp.float32),
                  jax.ShapeDtypeStruct((2, NP), jnp.float32)),
        mesh=_mesh(),
        compiler_params=_SC_PARAMS,
        scratch_types=[
            pltpu.VMEM((EDGES_PER_TILE,), jnp.int32),
            pltpu.VMEM((EDGES_PER_TILE,), jnp.int32),
            pltpu.VMEM((NP,), jnp.float32),
            pltpu.VMEM((NP,), jnp.float32),
            pltpu.VMEM((NP,), jnp.float32),
            pltpu.VMEM((16, ROWS_PER_TILE), jnp.float32),
            pltpu.VMEM((ROWS_PER_TILE,), jnp.float32),
            pltpu.VMEM((ROWS_PER_TILE,), jnp.float32),
            pltpu.VMEM((ROWS_PER_TILE,), jnp.float32),
            pltpu.VMEM_SHARED((16, NP), jnp.float32),
            pltpu.VMEM_SHARED((2, NP), jnp.float32),
        ],
    )(idx_a, feats_pad)


def _segsum_body(x_hbm, src_hbm, dst_hbm, agg_hbm,
                 sidx_v, didx_v, buf_v, acc_sh,
                 gsem0, gsem1, ssem0, ssem1, isem0, isem1):
    c = lax.axis_index("c")
    s = lax.axis_index("s")
    zeros = jnp.zeros((16,), jnp.float32)

    @pl.loop(0, CHUNK)
    def _(i):
        for k in range(H // 16):
            buf_v[0, i, pl.ds(k * 16, 16)] = zeros

    @pl.loop(0, ROWS_PER_TILE // CHUNK)
    def _(k):
        pltpu.async_copy(
            buf_v.at[0], acc_sh.at[pl.ds(s * ROWS_PER_TILE + k * CHUNK, CHUNK)],
            ssem0)

    pltpu.async_copy(src_hbm.at[s * NSTAGE], sidx_v.at[0], isem0)
    pltpu.async_copy(dst_hbm.at[s * NSTAGE], didx_v.at[0], isem1)

    @pl.loop(0, ROWS_PER_TILE // CHUNK)
    def _(k):
        pltpu.make_async_copy(
            buf_v.at[0], acc_sh.at[pl.ds(s * ROWS_PER_TILE + k * CHUNK, CHUNK)],
            ssem0).wait()

    plsc.subcore_barrier()

    xh = x_hbm.at[c]

    for st in range(NSTAGE):
        sl = st % 2
        sidx = sidx_v.at[sl]
        didx = didx_v.at[sl]
        pltpu.make_async_copy(src_hbm.at[s * NSTAGE + st], sidx, isem0).wait()
        pltpu.make_async_copy(dst_hbm.at[s * NSTAGE + st], didx, isem1).wait()
        if st + 1 < NSTAGE:
            pltpu.async_copy(src_hbm.at[s * NSTAGE + st + 1],
                             sidx_v.at[1 - sl], isem0)
            pltpu.async_copy(dst_hbm.at[s * NSTAGE + st + 1],
                             didx_v.at[1 - sl], isem1)
        pltpu.async_copy(xh.at[sidx.at[0]], buf_v.at[0], gsem0)
        pltpu.async_copy(xh.at[sidx.at[1]], buf_v.at[1], gsem1)

        def _phase(j, b, gsem, ssem, sidx=sidx, didx=didx):
            pltpu.make_async_copy(xh.at[sidx.at[j]], buf_v.at[b], gsem).wait()
            pltpu.async_copy(buf_v.at[b], acc_sh.at[didx.at[j]], ssem,
                             add=True)
            pltpu.make_async_copy(buf_v.at[b], acc_sh.at[didx.at[j]],
                                  ssem).wait()

            @pl.when(j + 2 < SCH)
            def _():
                pltpu.async_copy(xh.at[sidx.at[j + 2]], buf_v.at[b], gsem)

        @pl.loop(0, SCH // 2)
        def _(jj):
            _phase(jj * 2, 0, gsem0, ssem0)
            _phase(jj * 2 + 1, 1, gsem1, ssem1)

    plsc.subcore_barrier()
    pltpu.sync_copy(
        acc_sh.at[pl.ds(s * ROWS_PER_TILE, ROWS_PER_TILE)],
        agg_hbm.at[c].at[pl.ds(s * ROWS_PER_TILE, ROWS_PER_TILE)])


@jax.jit
def _segsum_call(x_st, src_b, dst_b):
    return pl.kernel(
        _segsum_body,
        out_type=jax.ShapeDtypeStruct((2, NP, H), jnp.float32),
        mesh=_mesh(),
        compiler_params=_SC_PARAMS,
        scratch_types=[
            pltpu.VMEM((2, SCH, CHUNK), jnp.int32),
            pltpu.VMEM((2, SCH, CHUNK), jnp.int32),
            pltpu.VMEM((2, CHUNK, H), jnp.float32),
            pltpu.VMEM_SHARED((NP, H), jnp.float32),
            pltpu.SemaphoreType.DMA,
            pltpu.SemaphoreType.DMA,
            pltpu.SemaphoreType.DMA,
            pltpu.SemaphoreType.DMA,
            pltpu.SemaphoreType.DMA,
            pltpu.SemaphoreType.DMA,
        ],
    )(x_st, src_b, dst_b)


def _dense_tail(last, i, a, w_ref, b_ref, g_ref, beta_ref, nsrc_ref, out_ref):
    t = jnp.dot(a.astype(jnp.bfloat16), w_ref[...].astype(jnp.bfloat16),
                preferred_element_type=jnp.float32)
    t = t + b_ref[0, :][None, :]
    mu = jnp.mean(t, axis=-1, keepdims=True)
    var = jnp.mean((t - mu) ** 2, axis=-1, keepdims=True)
    y = (t - mu) * lax.rsqrt(var + 1e-5) * g_ref[0, :][None, :] \
        + beta_ref[0, :][None, :]
    h = jnp.maximum(y, 0.0)
    rows = i * BLK + lax.broadcasted_iota(jnp.int32, (BLK, 1), 0)
    h = jnp.where(rows < N, h, 0.0)
    if last:
        out_ref[...] = h
    else:
        x = h * nsrc_ref[...]
        out_ref[0] = x[:, :H]
        out_ref[1] = x[:, H:]


def _layer1_body(a1_ref, a2_ref, ndst_ref, nsrc_ref, wseq_ref, bseq_ref,
                 w_ref, b_ref, g_ref, beta_ref, out_ref):
    i = pl.program_id(0)
    agg = a1_ref[...] * wseq_ref[0, :][None, :] \
        + a2_ref[...] * bseq_ref[0, :][None, :]
    a = agg * ndst_ref[...]
    _dense_tail(False, i, a, w_ref, b_ref, g_ref, beta_ref, nsrc_ref, out_ref)


@jax.jit
def _layer1_call(a1, a2, ndst, nsrc, wseq, bseq, w, b, g, beta):
    return pl.pallas_call(
        _layer1_body,
        grid=(NP // BLK,),
        in_specs=[
            pl.BlockSpec((BLK, 1), lambda i: (i, 0)),
            pl.BlockSpec((BLK, 1), lambda i: (i, 0)),
            pl.BlockSpec((BLK, 1), lambda i: (i, 0)),
            pl.BlockSpec((BLK, 1), lambda i: (i, 0)),
            pl.BlockSpec((1, D), lambda i: (0, 0)),
            pl.BlockSpec((1, D), lambda i: (0, 0)),
            pl.BlockSpec((D, D), lambda i: (0, 0)),
            pl.BlockSpec((1, D), lambda i: (0, 0)),
            pl.BlockSpec((1, D), lambda i: (0, 0)),
            pl.BlockSpec((1, D), lambda i: (0, 0)),
        ],
        out_specs=pl.BlockSpec((2, BLK, H), lambda i: (0, i, 0)),
        out_shape=jax.ShapeDtypeStruct((2, NP, H), jnp.float32),
    )(a1, a2, ndst, nsrc, wseq, bseq, w, b, g, beta)


def _layer_body(last, agg_ref, ndst_ref, nsrc_ref, w_ref, b_ref, g_ref,
                beta_ref, out_ref):
    i = pl.program_id(0)
    a = jnp.concatenate([agg_ref[0], agg_ref[1]], axis=-1)
    a = a * ndst_ref[...]
    _dense_tail(last, i, a, w_ref, b_ref, g_ref, beta_ref, nsrc_ref, out_ref)


@functools.partial(jax.jit, static_argnums=0)
def _layer_call(last, agg, ndst, nsrc, w, b, g, beta):
    if last:
        out_spec = pl.BlockSpec((BLK, D), lambda i: (i, 0))
        out_shape = jax.ShapeDtypeStruct((NP, D), jnp.float32)
    else:
        out_spec = pl.BlockSpec((2, BLK, H), lambda i: (0, i, 0))
        out_shape = jax.ShapeDtypeStruct((2, NP, H), jnp.float32)
    return pl.pallas_call(
        functools.partial(_layer_body, last),
        grid=(NP // BLK,),
        in_specs=[
            pl.BlockSpec((2, BLK, H), lambda i: (0, i, 0)),
            pl.BlockSpec((BLK, 1), lambda i: (i, 0)),
            pl.BlockSpec((BLK, 1), lambda i: (i, 0)),
            pl.BlockSpec((D, D), lambda i: (0, 0)),
            pl.BlockSpec((1, D), lambda i: (0, 0)),
            pl.BlockSpec((1, D), lambda i: (0, 0)),
            pl.BlockSpec((1, D), lambda i: (0, 0)),
        ],
        out_specs=out_spec,
        out_shape=out_shape,
    )(agg, ndst, nsrc, w, b, g, beta)


def _cls_body(x_ref, w_ref, b_ref, out_ref):
    k = pl.program_id(0)

    @pl.when(k == 0)
    def _():
        out_ref[...] = jnp.broadcast_to(b_ref[0, :][None, :], out_ref.shape)

    out_ref[...] += jnp.dot(x_ref[...], w_ref[...],
                            preferred_element_type=jnp.float32)


@jax.jit
def _cls_call(hr, w_cls, b_cls):
    kblk = D * 100 // 10
    return pl.pallas_call(
        _cls_body,
        grid=(10,),
        in_specs=[
            pl.BlockSpec((100, kblk), lambda k: (0, k)),
            pl.BlockSpec((kblk, NC_OUT), lambda k: (k, 0)),
            pl.BlockSpec((1, NC_OUT), lambda k: (0, 0)),
        ],
        out_specs=pl.BlockSpec((100, NC_OUT), lambda k: (0, 0)),
        out_shape=jax.ShapeDtypeStruct((100, NC_OUT), jnp.float32),
    )(hr, w_cls, b_cls)


def kernel(flow_feature, feats, edge_index, W_seq, b_seq,
           Wg0, bg0, g0, beta0, Wg1, bg1, g1, beta1, Wg2, bg2, g2, beta2,
           W_cls, b_cls):
    del flow_feature
    ei = jnp.concatenate(
        [edge_index, jnp.full((2, EP - E), N, jnp.int32)], axis=1)
    idx_a = ei.reshape(2, 16, EDGES_PER_TILE)
    src_b = ei[0].reshape(16 * NSTAGE, SCH, CHUNK)
    dst_b = ei[1].reshape(16 * NSTAGE, SCH, CHUNK)

    feats_pad = jnp.pad(feats, (0, NP - N))
    a12, nsd = _front_call(idx_a, feats_pad)
    nsrc = nsd[0].reshape(NP, 1)
    ndst = nsd[1].reshape(NP, 1)

    x_st = _layer1_call(a12[0].reshape(NP, 1), a12[1].reshape(NP, 1),
                        ndst, nsrc, W_seq, b_seq.reshape(1, D),
                        Wg0, bg0.reshape(1, D), g0.reshape(1, D),
                        beta0.reshape(1, D))

    h = None
    for l, (w, b, g, beta) in enumerate(
            [(Wg1, bg1, g1, beta1), (Wg2, bg2, g2, beta2)]):
        agg = _segsum_call(x_st, src_b, dst_b)
        res = _layer_call(l == 1, agg, ndst, nsrc, w, b.reshape(1, D),
                          g.reshape(1, D), beta.reshape(1, D))
        if l == 1:
            h = res
        else:
            x_st = res

    hr = h[:N].reshape(N // 100, D * 100)
    return _cls_call(hr, W_cls, b_cls.reshape(1, NC_OUT))


# speedup vs baseline: 1.0404x; 1.0404x over previous
import functools

import jax
import jax.numpy as jnp
from jax import lax
from jax.experimental import pallas as pl
from jax.experimental.pallas import tpu as pltpu
from jax.experimental.pallas import tpu_sc as plsc

N = 10000
NP = 10240
E = 160000
D = 256
H = 128
NC_OUT = 12
ROWS_PER_TILE = NP // 16
CHUNK = 128
NSTAGE = 4
SCH = 20
EDGES_PER_TILE = NSTAGE * SCH * CHUNK
EP = 16 * EDGES_PER_TILE
BLK = 1024

_SC_PARAMS = pltpu.CompilerParams(needs_layout_passes=False,
                                  internal_scratch_in_bytes=8192)


def _mesh():
    return plsc.VectorSubcoreMesh(core_axis_name="c", subcore_axis_name="s")


def _hist_reduce(s, hist_v, slots_sh, red_v, acc_v, out_row):
    pltpu.sync_copy(hist_v, slots_sh.at[s])
    plsc.subcore_barrier()
    pltpu.sync_copy(slots_sh.at[:, pl.ds(s * ROWS_PER_TILE, ROWS_PER_TILE)],
                    red_v)

    @pl.loop(0, ROWS_PER_TILE // 16)
    def _(j):
        v = red_v[0, pl.ds(j * 16, 16)]
        for r in range(1, 16):
            v = v + red_v[r, pl.ds(j * 16, 16)]
        acc_v[pl.ds(j * 16, 16)] = v

    pltpu.sync_copy(acc_v, out_row.at[pl.ds(s * ROWS_PER_TILE, ROWS_PER_TILE)])


def _zero_1d(ref, n):
    zeros = jnp.zeros((16,), jnp.float32)

    @pl.loop(0, n // 16)
    def _(i):
        ref[pl.ds(i * 16, 16)] = zeros


def _degree_body(idx_hbm, degs_hbm, idx_v, hist_v, slots_sh, red_v, acc_v):
    c = lax.axis_index("c")
    s = lax.axis_index("s")
    pltpu.sync_copy(idx_hbm.at[c, s], idx_v)
    _zero_1d(hist_v, NP)
    ones = jnp.ones((16,), jnp.float32)

    @pl.loop(0, EDGES_PER_TILE // 16)
    def _(i):
        ii = idx_v[pl.ds(i * 16, 16)]
        plsc.addupdate_scatter(hist_v, [ii], ones)

    _hist_reduce(s, hist_v, slots_sh, red_v, acc_v, degs_hbm.at[c])


@jax.jit
def _degree_call(idx_a):
    return pl.kernel(
        _degree_body,
        out_type=jax.ShapeDtypeStruct((2, NP), jnp.float32),
        mesh=_mesh(),
        compiler_params=_SC_PARAMS,
        scratch_types=[
            pltpu.VMEM((EDGES_PER_TILE,), jnp.int32),
            pltpu.VMEM((NP,), jnp.float32),
            pltpu.VMEM_SHARED((16, NP), jnp.float32),
            pltpu.VMEM((16, ROWS_PER_TILE), jnp.float32),
            pltpu.VMEM((ROWS_PER_TILE,), jnp.float32),
        ],
    )(idx_a)


def _pairseg_body(z_hbm, idx_hbm, out_hbm,
                  z_v, sidx_v, didx_v, hist_v, slots_sh, red_v, acc_v):
    c = lax.axis_index("c")
    s = lax.axis_index("s")
    pltpu.sync_copy(z_hbm.at[c], z_v)
    pltpu.sync_copy(idx_hbm.at[0, s], sidx_v)
    pltpu.sync_copy(idx_hbm.at[1, s], didx_v)
    _zero_1d(hist_v, NP)

    @pl.loop(0, EDGES_PER_TILE // 16)
    def _(i):
        sv = sidx_v[pl.ds(i * 16, 16)]
        dv = didx_v[pl.ds(i * 16, 16)]
        val = plsc.load_gather(z_v, [sv])
        plsc.addupdate_scatter(hist_v, [dv], val)

    _hist_reduce(s, hist_v, slots_sh, red_v, acc_v, out_hbm.at[c])


@jax.jit
def _pairseg_call(z, idx_a):
    return pl.kernel(
        _pairseg_body,
        out_type=jax.ShapeDtypeStruct((2, NP), jnp.float32),
        mesh=_mesh(),
        compiler_params=_SC_PARAMS,
        scratch_types=[
            pltpu.VMEM((NP,), jnp.float32),
            pltpu.VMEM((EDGES_PER_TILE,), jnp.int32),
            pltpu.VMEM((EDGES_PER_TILE,), jnp.int32),
            pltpu.VMEM((NP,), jnp.float32),
            pltpu.VMEM_SHARED((16, NP), jnp.float32),
            pltpu.VMEM((16, ROWS_PER_TILE), jnp.float32),
            pltpu.VMEM((ROWS_PER_TILE,), jnp.float32),
        ],
    )(z, idx_a)


def _segsum_body(x_hbm, src_hbm, dst_hbm, agg_hbm,
                 sidx_v, didx_v, buf_v, acc_sh,
                 gsem0, gsem1, ssem0, ssem1, isem0, isem1):
    c = lax.axis_index("c")
    s = lax.axis_index("s")
    zeros = jnp.zeros((16,), jnp.float32)

    @pl.loop(0, CHUNK)
    def _(i):
        for k in range(H // 16):
            buf_v[0, i, pl.ds(k * 16, 16)] = zeros

    @pl.loop(0, ROWS_PER_TILE // CHUNK)
    def _(k):
        pltpu.async_copy(
            buf_v.at[0], acc_sh.at[pl.ds(s * ROWS_PER_TILE + k * CHUNK, CHUNK)],
            ssem0)

    pltpu.async_copy(src_hbm.at[s * NSTAGE], sidx_v.at[0], isem0)
    pltpu.async_copy(dst_hbm.at[s * NSTAGE], didx_v.at[0], isem1)

    @pl.loop(0, ROWS_PER_TILE // CHUNK)
    def _(k):
        pltpu.make_async_copy(
            buf_v.at[0], acc_sh.at[pl.ds(s * ROWS_PER_TILE + k * CHUNK, CHUNK)],
            ssem0).wait()

    plsc.subcore_barrier()

    xh = x_hbm.at[c]

    for st in range(NSTAGE):
        sl = st % 2
        sidx = sidx_v.at[sl]
        didx = didx_v.at[sl]
        pltpu.make_async_copy(src_hbm.at[s * NSTAGE + st], sidx, isem0).wait()
        pltpu.make_async_copy(dst_hbm.at[s * NSTAGE + st], didx, isem1).wait()
        if st + 1 < NSTAGE:
            pltpu.async_copy(src_hbm.at[s * NSTAGE + st + 1],
                             sidx_v.at[1 - sl], isem0)
            pltpu.async_copy(dst_hbm.at[s * NSTAGE + st + 1],
                             didx_v.at[1 - sl], isem1)
        pltpu.async_copy(xh.at[sidx.at[0]], buf_v.at[0], gsem0)
        pltpu.async_copy(xh.at[sidx.at[1]], buf_v.at[1], gsem1)

        def _phase(j, b, gsem, ssem, sidx=sidx, didx=didx):
            pltpu.make_async_copy(xh.at[sidx.at[j]], buf_v.at[b], gsem).wait()
            pltpu.async_copy(buf_v.at[b], acc_sh.at[didx.at[j]], ssem,
                             add=True)
            pltpu.make_async_copy(buf_v.at[b], acc_sh.at[didx.at[j]],
                                  ssem).wait()

            @pl.when(j + 2 < SCH)
            def _():
                pltpu.async_copy(xh.at[sidx.at[j + 2]], buf_v.at[b], gsem)

        @pl.loop(0, SCH // 2)
        def _(jj):
            _phase(jj * 2, 0, gsem0, ssem0)
            _phase(jj * 2 + 1, 1, gsem1, ssem1)

    plsc.subcore_barrier()
    pltpu.sync_copy(
        acc_sh.at[pl.ds(s * ROWS_PER_TILE, ROWS_PER_TILE)],
        agg_hbm.at[c].at[pl.ds(s * ROWS_PER_TILE, ROWS_PER_TILE)])


@jax.jit
def _segsum_call(x_st, src_b, dst_b):
    return pl.kernel(
        _segsum_body,
        out_type=jax.ShapeDtypeStruct((2, NP, H), jnp.float32),
        mesh=_mesh(),
        compiler_params=_SC_PARAMS,
        scratch_types=[
            pltpu.VMEM((2, SCH, CHUNK), jnp.int32),
            pltpu.VMEM((2, SCH, CHUNK), jnp.int32),
            pltpu.VMEM((2, CHUNK, H), jnp.float32),
            pltpu.VMEM_SHARED((NP, H), jnp.float32),
            pltpu.SemaphoreType.DMA,
            pltpu.SemaphoreType.DMA,
            pltpu.SemaphoreType.DMA,
            pltpu.SemaphoreType.DMA,
            pltpu.SemaphoreType.DMA,
            pltpu.SemaphoreType.DMA,
        ],
    )(x_st, src_b, dst_b)


def _prep_body(dout_ref, din_ref, feats_ref, z_ref, nsrc_ref, ndst_ref):
    i = pl.program_id(0)
    ns = lax.rsqrt(jnp.maximum(dout_ref[...], 1.0))
    nd = lax.rsqrt(jnp.maximum(din_ref[...], 1.0))
    rows = i * BLK + lax.broadcasted_iota(jnp.int32, (BLK, 1), 0)
    valid = rows < N
    z_ref[0] = jnp.where(valid, feats_ref[...] * ns, 0.0)
    z_ref[1] = jnp.where(valid, ns, 0.0)
    nsrc_ref[...] = jnp.where(valid, ns, 0.0)
    ndst_ref[...] = jnp.where(valid, nd, 0.0)


@jax.jit
def _prep_call(dout, din, feats_p):
    return pl.pallas_call(
        _prep_body,
        grid=(NP // BLK,),
        in_specs=[
            pl.BlockSpec((BLK, 1), lambda i: (i, 0)),
            pl.BlockSpec((BLK, 1), lambda i: (i, 0)),
            pl.BlockSpec((BLK, 1), lambda i: (i, 0)),
        ],
        out_specs=[
            pl.BlockSpec((2, BLK, 1), lambda i: (0, i, 0)),
            pl.BlockSpec((BLK, 1), lambda i: (i, 0)),
            pl.BlockSpec((BLK, 1), lambda i: (i, 0)),
        ],
        out_shape=[
            jax.ShapeDtypeStruct((2, NP, 1), jnp.float32),
            jax.ShapeDtypeStruct((NP, 1), jnp.float32),
            jax.ShapeDtypeStruct((NP, 1), jnp.float32),
        ],
    )(dout, din, feats_p)


def _dense_tail(last, i, a, w_ref, b_ref, g_ref, beta_ref, nsrc_ref, out_ref):
    t = jnp.dot(a.astype(jnp.bfloat16), w_ref[...].astype(jnp.bfloat16),
                preferred_element_type=jnp.float32)
    t = t + b_ref[0, :][None, :]
    mu = jnp.mean(t, axis=-1, keepdims=True)
    var = jnp.mean((t - mu) ** 2, axis=-1, keepdims=True)
    y = (t - mu) * lax.rsqrt(var + 1e-5) * g_ref[0, :][None, :] \
        + beta_ref[0, :][None, :]
    h = jnp.maximum(y, 0.0)
    rows = i * BLK + lax.broadcasted_iota(jnp.int32, (BLK, 1), 0)
    h = jnp.where(rows < N, h, 0.0)
    if last:
        out_ref[...] = h
    else:
        x = h * nsrc_ref[...]
        out_ref[0] = x[:, :H]
        out_ref[1] = x[:, H:]


def _layer1_body(a1_ref, a2_ref, ndst_ref, nsrc_ref, wseq_ref, bseq_ref,
                 w_ref, b_ref, g_ref, beta_ref, out_ref):
    i = pl.program_id(0)
    agg = a1_ref[...] * wseq_ref[0, :][None, :] \
        + a2_ref[...] * bseq_ref[0, :][None, :]
    a = agg * ndst_ref[...]
    _dense_tail(False, i, a, w_ref, b_ref, g_ref, beta_ref, nsrc_ref, out_ref)


@jax.jit
def _layer1_call(a1, a2, ndst, nsrc, wseq, bseq, w, b, g, beta):
    return pl.pallas_call(
        _layer1_body,
        grid=(NP // BLK,),
        in_specs=[
            pl.BlockSpec((BLK, 1), lambda i: (i, 0)),
            pl.BlockSpec((BLK, 1), lambda i: (i, 0)),
            pl.BlockSpec((BLK, 1), lambda i: (i, 0)),
            pl.BlockSpec((BLK, 1), lambda i: (i, 0)),
            pl.BlockSpec((1, D), lambda i: (0, 0)),
            pl.BlockSpec((1, D), lambda i: (0, 0)),
            pl.BlockSpec((D, D), lambda i: (0, 0)),
            pl.BlockSpec((1, D), lambda i: (0, 0)),
            pl.BlockSpec((1, D), lambda i: (0, 0)),
            pl.BlockSpec((1, D), lambda i: (0, 0)),
        ],
        out_specs=pl.BlockSpec((2, BLK, H), lambda i: (0, i, 0)),
        out_shape=jax.ShapeDtypeStruct((2, NP, H), jnp.float32),
    )(a1, a2, ndst, nsrc, wseq, bseq, w, b, g, beta)


def _layer_body(last, agg_ref, ndst_ref, nsrc_ref, w_ref, b_ref, g_ref,
                beta_ref, out_ref):
    i = pl.program_id(0)
    a = jnp.concatenate([agg_ref[0], agg_ref[1]], axis=-1)
    a = a * ndst_ref[...]
    _dense_tail(last, i, a, w_ref, b_ref, g_ref, beta_ref, nsrc_ref, out_ref)


@functools.partial(jax.jit, static_argnums=0)
def _layer_call(last, agg, ndst, nsrc, w, b, g, beta):
    if last:
        out_spec = pl.BlockSpec((BLK, D), lambda i: (i, 0))
        out_shape = jax.ShapeDtypeStruct((NP, D), jnp.float32)
    else:
        out_spec = pl.BlockSpec((2, BLK, H), lambda i: (0, i, 0))
        out_shape = jax.ShapeDtypeStruct((2, NP, H), jnp.float32)
    return pl.pallas_call(
        functools.partial(_layer_body, last),
        grid=(NP // BLK,),
        in_specs=[
            pl.BlockSpec((2, BLK, H), lambda i: (0, i, 0)),
            pl.BlockSpec((BLK, 1), lambda i: (i, 0)),
            pl.BlockSpec((BLK, 1), lambda i: (i, 0)),
            pl.BlockSpec((D, D), lambda i: (0, 0)),
            pl.BlockSpec((1, D), lambda i: (0, 0)),
            pl.BlockSpec((1, D), lambda i: (0, 0)),
            pl.BlockSpec((1, D), lambda i: (0, 0)),
        ],
        out_specs=out_spec,
        out_shape=out_shape,
    )(agg, ndst, nsrc, w, b, g, beta)


def _cls_body(x_ref, w_ref, b_ref, out_ref):
    k = pl.program_id(0)

    @pl.when(k == 0)
    def _():
        out_ref[...] = jnp.broadcast_to(b_ref[0, :][None, :], out_ref.shape)

    out_ref[...] += jnp.dot(x_ref[...], w_ref[...],
                            preferred_element_type=jnp.float32)


@jax.jit
def _cls_call(hr, w_cls, b_cls):
    kblk = D * 100 // 10
    return pl.pallas_call(
        _cls_body,
        grid=(10,),
        in_specs=[
            pl.BlockSpec((100, kblk), lambda k: (0, k)),
            pl.BlockSpec((kblk, NC_OUT), lambda k: (k, 0)),
            pl.BlockSpec((1, NC_OUT), lambda k: (0, 0)),
        ],
        out_specs=pl.BlockSpec((100, NC_OUT), lambda k: (0, 0)),
        out_shape=jax.ShapeDtypeStruct((100, NC_OUT), jnp.float32),
    )(hr, w_cls, b_cls)


def kernel(flow_feature, feats, edge_index, W_seq, b_seq,
           Wg0, bg0, g0, beta0, Wg1, bg1, g1, beta1, Wg2, bg2, g2, beta2,
           W_cls, b_cls):
    del flow_feature
    ei = jnp.concatenate(
        [edge_index, jnp.full((2, EP - E), N, jnp.int32)], axis=1)
    idx_a = ei.reshape(2, 16, EDGES_PER_TILE)
    src_b = ei[0].reshape(16 * NSTAGE, SCH, CHUNK)
    dst_b = ei[1].reshape(16 * NSTAGE, SCH, CHUNK)

    degs = _degree_call(idx_a)
    feats_p = jnp.pad(feats, (0, NP - N)).reshape(NP, 1)
    z, nsrc, ndst = _prep_call(degs[0].reshape(NP, 1), degs[1].reshape(NP, 1),
                               feats_p)
    a12 = _pairseg_call(z.reshape(2, NP), idx_a)

    x_st = _layer1_call(a12[0].reshape(NP, 1), a12[1].reshape(NP, 1),
                        ndst, nsrc, W_seq, b_seq.reshape(1, D),
                        Wg0, bg0.reshape(1, D), g0.reshape(1, D),
                        beta0.reshape(1, D))

    h = None
    for l, (w, b, g, beta) in enumerate(
            [(Wg1, bg1, g1, beta1), (Wg2, bg2, g2, beta2)]):
        agg = _segsum_call(x_st, src_b, dst_b)
        res = _layer_call(l == 1, agg, ndst, nsrc, w, b.reshape(1, D),
                          g.reshape(1, D), beta.reshape(1, D))
        if l == 1:
            h = res
        else:
            x_st = res

    hr = h[:N].reshape(N // 100, D * 100)
    return _cls_call(hr, W_cls, b_cls.reshape(1, NC_OUT))
